# baseline (speedup 1.0000x reference)
import functools

import jax
import jax.numpy as jnp
from jax import lax
from jax.experimental import pallas as pl
from jax.experimental.pallas import tpu as pltpu
from jax.experimental.pallas import tpu_sc as plsc

N_NODES = 10000
N_EDGES = 320000
IN_C = 128
HID = 128
OUT_C = 64

NC = 2
NS = 16
NW = NC * NS
CHUNK = 128
NCHUNK = 80
EPW = NCHUNK * CHUNK
E_PAD = NW * EPW
NPAD = 10240
SLAB = NPAD // NS
N_SINK = NPAD - N_NODES

_mesh = plsc.VectorSubcoreMesh(core_axis_name="c", subcore_axis_name="s")


@functools.partial(
    pl.kernel,
    out_type=jax.ShapeDtypeStruct((NC, NPAD), jnp.float32),
    mesh=_mesh,
    scratch_types=[
        pltpu.VMEM((NCHUNK, CHUNK), jnp.int32),
        pltpu.VMEM((CHUNK,), jnp.float32),
        pltpu.VMEM_SHARED((NPAD,), jnp.float32),
        pltpu.SemaphoreType.DMA,
    ],
)
def _deg_kernel(dst_hbm, zeros_hbm, deg_out, didx, ones_v, deg_sh, sem):
    c = lax.axis_index("c")
    s = lax.axis_index("s")
    w = c * NS + s
    pltpu.sync_copy(zeros_hbm.at[pl.ds(s * SLAB, SLAB)],
                    deg_sh.at[pl.ds(s * SLAB, SLAB)])
    for j in range(CHUNK // 16):
        ones_v[pl.ds(j * 16, 16)] = jnp.ones((16,), jnp.float32)
    pltpu.sync_copy(dst_hbm.at[w], didx)
    plsc.subcore_barrier()

    def issue(i, carry):
        pltpu.async_copy(ones_v, deg_sh.at[didx.at[i]], sem, add=True)
        return carry

    lax.fori_loop(0, NCHUNK, issue, 0)

    def drain(i, carry):
        pltpu.make_async_copy(ones_v, deg_sh.at[didx.at[i]], sem).wait()
        return carry

    lax.fori_loop(0, NCHUNK, drain, 0)
    plsc.subcore_barrier()
    pltpu.sync_copy(deg_sh.at[pl.ds(s * SLAB, SLAB)],
                    deg_out.at[c, pl.ds(s * SLAB, SLAB)])


def _make_agg_kernel(d_feat):
    @functools.partial(
        pl.kernel,
        out_type=jax.ShapeDtypeStruct((NC, NPAD, d_feat), jnp.float32),
        mesh=_mesh,
        scratch_types=[
            pltpu.VMEM((2, 2, CHUNK), jnp.int32),
            pltpu.VMEM((2, CHUNK, d_feat), jnp.float32),
            pltpu.VMEM_SHARED((NPAD, d_feat), jnp.float32),
            pltpu.SemaphoreType.DMA,
            pltpu.SemaphoreType.DMA,
            pltpu.SemaphoreType.DMA,
            pltpu.SemaphoreType.DMA,
        ],
        compiler_params=pltpu.CompilerParams(use_tc_tiling_on_sc=False),
    )
    def agg_kernel(g_hbm, idx_hbm, zeros_hbm, acc_out,
                   ibuf, rows, acc_sh, is0, is1, gs0, gs1):
        c = lax.axis_index("c")
        s = lax.axis_index("s")
        w = c * NS + s
        isems = (is0, is1)
        gsems = (gs0, gs1)

        def iload(j, b):
            pltpu.async_copy(idx_hbm.at[w, j], ibuf.at[b], isems[b])

        def iwait(b):
            pltpu.make_async_copy(idx_hbm.at[w, 0], ibuf.at[b],
                                  isems[b]).wait()

        def gather(b_idx, b_rows):
            pltpu.async_copy(g_hbm.at[ibuf.at[b_idx, 0]], rows.at[b_rows],
                             gsems[b_rows])

        def gwait(b):
            pltpu.make_async_copy(g_hbm.at[ibuf.at[0, 0]], rows.at[b],
                                  gsems[b]).wait()

        iload(0, 0)
        iload(1, 1)
        pltpu.sync_copy(zeros_hbm.at[pl.ds(s * SLAB, SLAB)],
                        acc_sh.at[pl.ds(s * SLAB, SLAB)])
        plsc.subcore_barrier()
        iwait(0)
        gather(0, 0)

        def slot(j, b):
            iwait(1 - b)
            gwait(b)
            gather(1 - b, 1 - b)
            pltpu.sync_copy(rows.at[b], acc_sh.at[ibuf.at[b, 1]], add=True)
            iload(jnp.minimum(j + 2, NCHUNK - 1), b)

        def body(t, carry):
            slot(2 * t, 0)
            slot(2 * t + 1, 1)
            return carry

        lax.fori_loop(0, NCHUNK // 2, body, 0)
        gwait(0)
        iwait(1)
        plsc.subcore_barrier()
        pltpu.sync_copy(acc_sh.at[pl.ds(s * SLAB, SLAB)],
                        acc_out.at[c, pl.ds(s * SLAB, SLAB)])

    return agg_kernel


_agg128 = _make_agg_kernel(HID)
_agg64 = _make_agg_kernel(OUT_C)


BN = 1000


def _t1_body(x_ref, w_ref, dega_ref, degb_ref, g_ref, dis_ref):
    deg = dega_ref[...] + degb_ref[...] + 1.0
    dis = lax.rsqrt(deg)
    g_ref[...] = dis * jnp.dot(x_ref[...], w_ref[...],
                               preferred_element_type=jnp.float32)
    dis_ref[...] = dis


def _t2_body(acc_ref, g1_ref, dis_ref, b_ref, w_ref, g2_ref):
    dis = dis_ref[...]
    h = dis * (acc_ref[0] + acc_ref[1] + g1_ref[...]) + b_ref[...]
    h = jnp.maximum(h, 0.0)
    g2_ref[...] = dis * jnp.dot(h, w_ref[...],
                                preferred_element_type=jnp.float32)


def _t3_body(acc_ref, g2_ref, dis_ref, b_ref, out_ref):
    out_ref[...] = (dis_ref[...] * (acc_ref[0] + acc_ref[1] + g2_ref[...])
                    + b_ref[...])


def kernel(x, edge_index, W1, b1, W2, b2):
    src = edge_index[0].astype(jnp.int32)
    dst = edge_index[1].astype(jnp.int32)
    ppw = EPW - N_EDGES // NW
    pad_src = jnp.zeros((NW, ppw), jnp.int32)
    pad_dst = jnp.broadcast_to(
        N_NODES + (jnp.arange(ppw, dtype=jnp.int32) % N_SINK), (NW, ppw))
    src3 = jnp.concatenate(
        [src.reshape(NW, -1), pad_src], axis=1).reshape(NW, NCHUNK, CHUNK)
    dst3 = jnp.concatenate(
        [dst.reshape(NW, -1), pad_dst], axis=1).reshape(NW, NCHUNK, CHUNK)
    idx4 = jnp.stack([src3, dst3], axis=2)
    zeros1d = jnp.zeros((NPAD,), jnp.float32)
    zeros_h = jnp.zeros((NPAD, HID), jnp.float32)
    zeros_o = jnp.zeros((NPAD, OUT_C), jnp.float32)

    deg_parts = _deg_kernel(dst3, zeros1d)
    dega = deg_parts[0, :N_NODES, None]
    degb = deg_parts[1, :N_NODES, None]

    grid = (N_NODES // BN,)
    g1, dis = pl.pallas_call(
        _t1_body,
        grid=grid,
        in_specs=[
            pl.BlockSpec((BN, IN_C), lambda i: (i, 0)),
            pl.BlockSpec((IN_C, HID), lambda i: (0, 0)),
            pl.BlockSpec((BN, 1), lambda i: (i, 0)),
            pl.BlockSpec((BN, 1), lambda i: (i, 0)),
        ],
        out_specs=[
            pl.BlockSpec((BN, HID), lambda i: (i, 0)),
            pl.BlockSpec((BN, 1), lambda i: (i, 0)),
        ],
        out_shape=[
            jax.ShapeDtypeStruct((N_NODES, HID), jnp.float32),
            jax.ShapeDtypeStruct((N_NODES, 1), jnp.float32),
        ],
    )(x, W1, dega, degb)

    acc1 = _agg128(g1, idx4, zeros_h)[:, :N_NODES, :]

    g2 = pl.pallas_call(
        _t2_body,
        grid=grid,
        in_specs=[
            pl.BlockSpec((NC, BN, HID), lambda i: (0, i, 0)),
            pl.BlockSpec((BN, HID), lambda i: (i, 0)),
            pl.BlockSpec((BN, 1), lambda i: (i, 0)),
            pl.BlockSpec((1, HID), lambda i: (0, 0)),
            pl.BlockSpec((HID, OUT_C), lambda i: (0, 0)),
        ],
        out_specs=pl.BlockSpec((BN, OUT_C), lambda i: (i, 0)),
        out_shape=jax.ShapeDtypeStruct((N_NODES, OUT_C), jnp.float32),
    )(acc1, g1, dis, b1[None, :], W2)

    acc2 = _agg64(g2, idx4, zeros_o)[:, :N_NODES, :]

    out = pl.pallas_call(
        _t3_body,
        grid=grid,
        in_specs=[
            pl.BlockSpec((NC, BN, OUT_C), lambda i: (0, i, 0)),
            pl.BlockSpec((BN, OUT_C), lambda i: (i, 0)),
            pl.BlockSpec((BN, 1), lambda i: (i, 0)),
            pl.BlockSpec((1, OUT_C), lambda i: (0, 0)),
        ],
        out_specs=pl.BlockSpec((BN, OUT_C), lambda i: (i, 0)),
        out_shape=jax.ShapeDtypeStruct((N_NODES, OUT_C), jnp.float32),
    )(acc2, g2, dis, b2[None, :])

    return out


# speedup vs baseline: 2.9847x; 2.9847x over previous
import functools

import jax
import jax.numpy as jnp
from jax import lax
from jax.experimental import pallas as pl
from jax.experimental.pallas import tpu as pltpu
from jax.experimental.pallas import tpu_sc as plsc

N_NODES = 10000
N_EDGES = 320000
IN_C = 128
HID = 128
OUT_C = 64

NC = 2
NS = 16
NW = NC * NS
CHUNK = 128
NCHUNK = 80
EPW = NCHUNK * CHUNK
REAL_PW = N_EDGES // NW
PAD_PW = EPW - REAL_PW
FULL_CHUNKS = REAL_PW // CHUNK
TAIL_REAL = REAL_PW - FULL_CHUNKS * CHUNK
NPAD = 10240
SLAB = NPAD // NS

_mesh = plsc.VectorSubcoreMesh(core_axis_name="c", subcore_axis_name="s")


@functools.partial(
    pl.kernel,
    out_type=jax.ShapeDtypeStruct((NC, NPAD), jnp.float32),
    mesh=_mesh,
    scratch_types=[
        pltpu.VMEM((NCHUNK, CHUNK), jnp.int32),
        pltpu.VMEM((CHUNK,), jnp.float32),
        pltpu.VMEM((CHUNK,), jnp.float32),
        pltpu.VMEM_SHARED((NPAD,), jnp.float32),
        pltpu.SemaphoreType.DMA,
    ],
)
def _deg_kernel(dst_hbm, zeros_hbm, deg_out, didx, ones_v, tail_v, deg_sh,
                sem):
    c = lax.axis_index("c")
    s = lax.axis_index("s")
    w = c * NS + s
    pltpu.sync_copy(zeros_hbm.at[pl.ds(s * SLAB, SLAB)],
                    deg_sh.at[pl.ds(s * SLAB, SLAB)])
    for j in range(CHUNK // 16):
        ones_v[pl.ds(j * 16, 16)] = jnp.ones((16,), jnp.float32)
        tail_v[pl.ds(j * 16, 16)] = jnp.full(
            (16,), 1.0 if (j + 1) * 16 <= TAIL_REAL else 0.0, jnp.float32)
    pltpu.sync_copy(dst_hbm.at[w], didx)
    plsc.subcore_barrier()

    def issue(i, carry):
        pltpu.async_copy(ones_v, deg_sh.at[didx.at[i]], sem, add=True)
        return carry

    lax.fori_loop(0, FULL_CHUNKS, issue, 0)
    pltpu.async_copy(tail_v, deg_sh.at[didx.at[FULL_CHUNKS]], sem, add=True)

    def drain(i, carry):
        pltpu.make_async_copy(ones_v, deg_sh.at[didx.at[i]], sem).wait()
        return carry

    lax.fori_loop(0, FULL_CHUNKS + 1, drain, 0)
    plsc.subcore_barrier()
    pltpu.sync_copy(deg_sh.at[pl.ds(s * SLAB, SLAB)],
                    deg_out.at[c, pl.ds(s * SLAB, SLAB)])


def _make_agg_kernel(d_feat):
    @functools.partial(
        pl.kernel,
        out_type=jax.ShapeDtypeStruct((NC, NPAD, d_feat), jnp.float32),
        mesh=_mesh,
        scratch_types=[
            pltpu.VMEM((2, 2, CHUNK), jnp.int32),
            pltpu.VMEM((2, CHUNK, d_feat), jnp.float32),
            pltpu.VMEM_SHARED((NPAD, d_feat), jnp.float32),
            pltpu.SemaphoreType.DMA,
            pltpu.SemaphoreType.DMA,
            pltpu.SemaphoreType.DMA,
            pltpu.SemaphoreType.DMA,
        ],
        compiler_params=pltpu.CompilerParams(use_tc_tiling_on_sc=False),
    )
    def agg_kernel(g_hbm, idx_hbm, zeros_hbm, acc_out,
                   ibuf, rows, acc_sh, is0, is1, gs0, gs1):
        c = lax.axis_index("c")
        s = lax.axis_index("s")
        w = c * NS + s
        isems = (is0, is1)
        gsems = (gs0, gs1)

        def iload(j, b):
            pltpu.async_copy(idx_hbm.at[w, j], ibuf.at[b], isems[b])

        def iwait(b):
            pltpu.make_async_copy(idx_hbm.at[w, 0], ibuf.at[b],
                                  isems[b]).wait()

        def gather(b_idx, b_rows):
            pltpu.async_copy(g_hbm.at[ibuf.at[b_idx, 0]], rows.at[b_rows],
                             gsems[b_rows])

        def gwait(b):
            pltpu.make_async_copy(g_hbm.at[ibuf.at[0, 0]], rows.at[b],
                                  gsems[b]).wait()

        iload(0, 0)
        iload(1, 1)
        pltpu.sync_copy(zeros_hbm.at[pl.ds(s * SLAB, SLAB)],
                        acc_sh.at[pl.ds(s * SLAB, SLAB)])
        plsc.subcore_barrier()
        iwait(0)
        gather(0, 0)

        def slot(j, b):
            iwait(1 - b)
            gwait(b)
            gather(1 - b, 1 - b)
            pltpu.sync_copy(rows.at[b], acc_sh.at[ibuf.at[b, 1]], add=True)
            iload(jnp.minimum(j + 2, NCHUNK - 1), b)

        def body(t, carry):
            slot(2 * t, 0)
            slot(2 * t + 1, 1)
            return carry

        lax.fori_loop(0, NCHUNK // 2, body, 0)
        gwait(0)
        iwait(1)
        plsc.subcore_barrier()
        pltpu.sync_copy(acc_sh.at[pl.ds(s * SLAB, SLAB)],
                        acc_out.at[c, pl.ds(s * SLAB, SLAB)])

    return agg_kernel


_agg128 = _make_agg_kernel(HID)
_agg64 = _make_agg_kernel(OUT_C)


BN = 1024


def _t1_body(x_ref, w_ref, dega_ref, degb_ref, g_ref, dis_ref):
    i = pl.program_id(0)
    row = i * BN + lax.broadcasted_iota(jnp.int32, (BN, 1), 0)
    deg = dega_ref[...] + degb_ref[...] + 1.0
    dis = jnp.where(row < N_NODES, lax.rsqrt(deg), 0.0)
    g_ref[...] = dis * jnp.dot(x_ref[...], w_ref[...],
                               preferred_element_type=jnp.float32)
    dis_ref[...] = dis


def _t2_body(acc_ref, g1_ref, dis_ref, b_ref, w_ref, g2_ref):
    dis = dis_ref[...]
    h = dis * (acc_ref[0] + acc_ref[1] + g1_ref[...]) + b_ref[...]
    h = jnp.maximum(h, 0.0)
    g2_ref[...] = dis * jnp.dot(h, w_ref[...],
                                preferred_element_type=jnp.float32)


def _t3_body(acc_ref, g2_ref, dis_ref, b_ref, out_ref):
    out_ref[...] = (dis_ref[...] * (acc_ref[0] + acc_ref[1] + g2_ref[...])
                    + b_ref[...])


def kernel(x, edge_index, W1, b1, W2, b2):
    src = edge_index[0].astype(jnp.int32)
    dst = edge_index[1].astype(jnp.int32)
    warange = jnp.arange(PAD_PW, dtype=jnp.int32)
    pad_src = jnp.broadcast_to(N_NODES + (warange % (NPAD - N_NODES)),
                               (NW, PAD_PW))
    pad_dst = (jnp.arange(NW, dtype=jnp.int32)[:, None] * PAD_PW
               + warange[None, :]) % N_NODES
    src3 = jnp.concatenate(
        [src.reshape(NW, -1), pad_src], axis=1).reshape(NW, NCHUNK, CHUNK)
    dst3 = jnp.concatenate(
        [dst.reshape(NW, -1), pad_dst], axis=1).reshape(NW, NCHUNK, CHUNK)
    idx4 = jnp.stack([src3, dst3], axis=2)
    x_pad = jnp.concatenate(
        [x, jnp.zeros((NPAD - N_NODES, IN_C), jnp.float32)], axis=0)
    zeros1d = jnp.zeros((NPAD,), jnp.float32)
    zeros_h = jnp.zeros((NPAD, HID), jnp.float32)
    zeros_o = jnp.zeros((NPAD, OUT_C), jnp.float32)

    deg_parts = _deg_kernel(dst3, zeros1d)
    dega = deg_parts[0, :, None]
    degb = deg_parts[1, :, None]

    grid = (NPAD // BN,)
    g1, dis = pl.pallas_call(
        _t1_body,
        grid=grid,
        in_specs=[
            pl.BlockSpec((BN, IN_C), lambda i: (i, 0)),
            pl.BlockSpec((IN_C, HID), lambda i: (0, 0)),
            pl.BlockSpec((BN, 1), lambda i: (i, 0)),
            pl.BlockSpec((BN, 1), lambda i: (i, 0)),
        ],
        out_specs=[
            pl.BlockSpec((BN, HID), lambda i: (i, 0)),
            pl.BlockSpec((BN, 1), lambda i: (i, 0)),
        ],
        out_shape=[
            jax.ShapeDtypeStruct((NPAD, HID), jnp.float32),
            jax.ShapeDtypeStruct((NPAD, 1), jnp.float32),
        ],
    )(x_pad, W1, dega, degb)

    acc1 = _agg128(g1, idx4, zeros_h)

    g2 = pl.pallas_call(
        _t2_body,
        grid=grid,
        in_specs=[
            pl.BlockSpec((NC, BN, HID), lambda i: (0, i, 0)),
            pl.BlockSpec((BN, HID), lambda i: (i, 0)),
            pl.BlockSpec((BN, 1), lambda i: (i, 0)),
            pl.BlockSpec((1, HID), lambda i: (0, 0)),
            pl.BlockSpec((HID, OUT_C), lambda i: (0, 0)),
        ],
        out_specs=pl.BlockSpec((BN, OUT_C), lambda i: (i, 0)),
        out_shape=jax.ShapeDtypeStruct((NPAD, OUT_C), jnp.float32),
    )(acc1, g1, dis, b1[None, :], W2)

    acc2 = _agg64(g2, idx4, zeros_o)

    out = pl.pallas_call(
        _t3_body,
        grid=grid,
        in_specs=[
            pl.BlockSpec((NC, BN, OUT_C), lambda i: (0, i, 0)),
            pl.BlockSpec((BN, OUT_C), lambda i: (i, 0)),
            pl.BlockSpec((BN, 1), lambda i: (i, 0)),
            pl.BlockSpec((1, OUT_C), lambda i: (0, 0)),
        ],
        out_specs=pl.BlockSpec((BN, OUT_C), lambda i: (i, 0)),
        out_shape=jax.ShapeDtypeStruct((NPAD, OUT_C), jnp.float32),
    )(acc2, g2, dis, b2[None, :])

    return out[:N_NODES]


# speedup vs baseline: 3.0091x; 1.0082x over previous
import functools

import jax
import jax.numpy as jnp
from jax import lax
from jax.experimental import pallas as pl
from jax.experimental.pallas import tpu as pltpu
from jax.experimental.pallas import tpu_sc as plsc

N_NODES = 10000
N_EDGES = 320000
IN_C = 128
HID = 128
OUT_C = 64

NC = 2
NS = 16
NW = NC * NS
CHUNK = 128
NCHUNK = 80
EPW = NCHUNK * CHUNK
REAL_PW = N_EDGES // NW
PAD_PW = EPW - REAL_PW
FULL_CHUNKS = REAL_PW // CHUNK
TAIL_REAL = REAL_PW - FULL_CHUNKS * CHUNK
NPAD = 10240
SLAB = NPAD // NS

_mesh = plsc.VectorSubcoreMesh(core_axis_name="c", subcore_axis_name="s")


@functools.partial(
    pl.kernel,
    out_type=jax.ShapeDtypeStruct((NC, NPAD), jnp.float32),
    mesh=_mesh,
    scratch_types=[
        pltpu.VMEM((NCHUNK, CHUNK), jnp.int32),
        pltpu.VMEM((CHUNK,), jnp.float32),
        pltpu.VMEM((CHUNK,), jnp.float32),
        pltpu.VMEM_SHARED((NPAD,), jnp.float32),
        pltpu.SemaphoreType.DMA,
    ],
)
def _deg_kernel(dst_hbm, zeros_hbm, deg_out, didx, ones_v, tail_v, deg_sh,
                sem):
    c = lax.axis_index("c")
    s = lax.axis_index("s")
    w = c * NS + s
    pltpu.sync_copy(zeros_hbm.at[pl.ds(s * SLAB, SLAB)],
                    deg_sh.at[pl.ds(s * SLAB, SLAB)])
    for j in range(CHUNK // 16):
        ones_v[pl.ds(j * 16, 16)] = jnp.ones((16,), jnp.float32)
        tail_v[pl.ds(j * 16, 16)] = jnp.full(
            (16,), 1.0 if (j + 1) * 16 <= TAIL_REAL else 0.0, jnp.float32)
    pltpu.sync_copy(dst_hbm.at[w], didx)
    plsc.subcore_barrier()

    def issue(i, carry):
        pltpu.async_copy(ones_v, deg_sh.at[didx.at[i]], sem, add=True)
        return carry

    lax.fori_loop(0, FULL_CHUNKS, issue, 0)
    pltpu.async_copy(tail_v, deg_sh.at[didx.at[FULL_CHUNKS]], sem, add=True)

    def drain(i, carry):
        pltpu.make_async_copy(ones_v, deg_sh.at[didx.at[i]], sem).wait()
        return carry

    lax.fori_loop(0, FULL_CHUNKS + 1, drain, 0)
    plsc.subcore_barrier()
    pltpu.sync_copy(deg_sh.at[pl.ds(s * SLAB, SLAB)],
                    deg_out.at[c, pl.ds(s * SLAB, SLAB)])


def _make_agg_kernel(d_feat):
    @functools.partial(
        pl.kernel,
        out_type=jax.ShapeDtypeStruct((NC, NPAD, d_feat), jnp.float32),
        mesh=_mesh,
        scratch_types=[
            pltpu.VMEM((4, 2, CHUNK), jnp.int32),
            pltpu.VMEM((2, CHUNK, d_feat), jnp.float32),
            pltpu.VMEM_SHARED((NPAD, d_feat), jnp.float32),
            pltpu.SemaphoreType.DMA,
            pltpu.SemaphoreType.DMA,
            pltpu.SemaphoreType.DMA,
            pltpu.SemaphoreType.DMA,
            pltpu.SemaphoreType.DMA,
            pltpu.SemaphoreType.DMA,
            pltpu.SemaphoreType.DMA,
            pltpu.SemaphoreType.DMA,
        ],
        compiler_params=pltpu.CompilerParams(use_tc_tiling_on_sc=False),
    )
    def agg_kernel(g_hbm, idx_hbm, zeros_hbm, acc_out,
                   ibuf, rows, acc_sh, is0, is1, is2, is3, gs0, gs1, ss0, ss1):
        c = lax.axis_index("c")
        s = lax.axis_index("s")
        w = c * NS + s
        isems = (is0, is1, is2, is3)
        gsems = (gs0, gs1)
        ssems = (ss0, ss1)

        def iload(j, q):
            pltpu.async_copy(idx_hbm.at[w, j], ibuf.at[q], isems[q])

        def iwait(q):
            pltpu.make_async_copy(idx_hbm.at[w, 0], ibuf.at[q],
                                  isems[q]).wait()

        def gather(q, b):
            pltpu.async_copy(g_hbm.at[ibuf.at[q, 0]], rows.at[b], gsems[b])

        def gwait(b):
            pltpu.make_async_copy(g_hbm.at[ibuf.at[0, 0]], rows.at[b],
                                  gsems[b]).wait()

        def scat(q, b):
            pltpu.async_copy(rows.at[b], acc_sh.at[ibuf.at[q, 1]], ssems[b],
                             add=True)

        def swait(b):
            pltpu.make_async_copy(rows.at[b], acc_sh.at[ibuf.at[0, 1]],
                                  ssems[b]).wait()

        def slot(j, k, first=False, gnext=True, lnext=True):
            q, b = k % 4, k % 2
            if gnext:
                iwait((k + 1) % 4)
            gwait(b)
            scat(q, b)
            if not first:
                swait(1 - b)
            if gnext:
                gather((k + 1) % 4, 1 - b)
            if lnext:
                iload(j + 3, (k + 3) % 4)

        iload(0, 0)
        iload(1, 1)
        iload(2, 2)
        pltpu.sync_copy(zeros_hbm.at[pl.ds(s * SLAB, SLAB)],
                        acc_sh.at[pl.ds(s * SLAB, SLAB)])
        plsc.subcore_barrier()
        iwait(0)
        gather(0, 0)
        slot(0, 0, first=True)
        slot(1, 1)
        slot(2, 2)
        slot(3, 3)

        def body(t, carry):
            j = 4 * t
            slot(j, 0)
            slot(j + 1, 1)
            slot(j + 2, 2)
            slot(j + 3, 3)
            return carry

        lax.fori_loop(1, NCHUNK // 4 - 1, body, 0)
        slot(NCHUNK - 4, 0, lnext=True)
        slot(NCHUNK - 3, 1, lnext=False)
        slot(NCHUNK - 2, 2, lnext=False)
        slot(NCHUNK - 1, 3, gnext=False, lnext=False)
        swait(1)
        plsc.subcore_barrier()
        pltpu.sync_copy(acc_sh.at[pl.ds(s * SLAB, SLAB)],
                        acc_out.at[c, pl.ds(s * SLAB, SLAB)])

    return agg_kernel


_agg128 = _make_agg_kernel(HID)
_agg64 = _make_agg_kernel(OUT_C)


BN = 1024


def _t1_body(x_ref, w_ref, dega_ref, degb_ref, g_ref, dis_ref):
    i = pl.program_id(0)
    row = i * BN + lax.broadcasted_iota(jnp.int32, (BN, 1), 0)
    deg = dega_ref[...] + degb_ref[...] + 1.0
    dis = jnp.where(row < N_NODES, lax.rsqrt(deg), 0.0)
    g_ref[...] = dis * jnp.dot(x_ref[...], w_ref[...],
                               preferred_element_type=jnp.float32)
    dis_ref[...] = dis


def _t2_body(acc_ref, g1_ref, dis_ref, b_ref, w_ref, g2_ref):
    dis = dis_ref[...]
    h = dis * (acc_ref[0] + acc_ref[1] + g1_ref[...]) + b_ref[...]
    h = jnp.maximum(h, 0.0)
    g2_ref[...] = dis * jnp.dot(h, w_ref[...],
                                preferred_element_type=jnp.float32)


def _t3_body(acc_ref, g2_ref, dis_ref, b_ref, out_ref):
    out_ref[...] = (dis_ref[...] * (acc_ref[0] + acc_ref[1] + g2_ref[...])
                    + b_ref[...])


def kernel(x, edge_index, W1, b1, W2, b2):
    src = edge_index[0].astype(jnp.int32)
    dst = edge_index[1].astype(jnp.int32)
    warange = jnp.arange(PAD_PW, dtype=jnp.int32)
    pad_src = jnp.broadcast_to(N_NODES + (warange % (NPAD - N_NODES)),
                               (NW, PAD_PW))
    pad_dst = (jnp.arange(NW, dtype=jnp.int32)[:, None] * PAD_PW
               + warange[None, :]) % N_NODES
    src3 = jnp.concatenate(
        [src.reshape(NW, -1), pad_src], axis=1).reshape(NW, NCHUNK, CHUNK)
    dst3 = jnp.concatenate(
        [dst.reshape(NW, -1), pad_dst], axis=1).reshape(NW, NCHUNK, CHUNK)
    idx4 = jnp.stack([src3, dst3], axis=2)
    x_pad = jnp.concatenate(
        [x, jnp.zeros((NPAD - N_NODES, IN_C), jnp.float32)], axis=0)
    zeros1d = jnp.zeros((NPAD,), jnp.float32)
    zeros_h = jnp.zeros((NPAD, HID), jnp.float32)
    zeros_o = jnp.zeros((NPAD, OUT_C), jnp.float32)

    deg_parts = _deg_kernel(dst3, zeros1d)
    dega = deg_parts[0, :, None]
    degb = deg_parts[1, :, None]

    grid = (NPAD // BN,)
    g1, dis = pl.pallas_call(
        _t1_body,
        grid=grid,
        in_specs=[
            pl.BlockSpec((BN, IN_C), lambda i: (i, 0)),
            pl.BlockSpec((IN_C, HID), lambda i: (0, 0)),
            pl.BlockSpec((BN, 1), lambda i: (i, 0)),
            pl.BlockSpec((BN, 1), lambda i: (i, 0)),
        ],
        out_specs=[
            pl.BlockSpec((BN, HID), lambda i: (i, 0)),
            pl.BlockSpec((BN, 1), lambda i: (i, 0)),
        ],
        out_shape=[
            jax.ShapeDtypeStruct((NPAD, HID), jnp.float32),
            jax.ShapeDtypeStruct((NPAD, 1), jnp.float32),
        ],
    )(x_pad, W1, dega, degb)

    acc1 = _agg128(g1, idx4, zeros_h)

    g2 = pl.pallas_call(
        _t2_body,
        grid=grid,
        in_specs=[
            pl.BlockSpec((NC, BN, HID), lambda i: (0, i, 0)),
            pl.BlockSpec((BN, HID), lambda i: (i, 0)),
            pl.BlockSpec((BN, 1), lambda i: (i, 0)),
            pl.BlockSpec((1, HID), lambda i: (0, 0)),
            pl.BlockSpec((HID, OUT_C), lambda i: (0, 0)),
        ],
        out_specs=pl.BlockSpec((BN, OUT_C), lambda i: (i, 0)),
        out_shape=jax.ShapeDtypeStruct((NPAD, OUT_C), jnp.float32),
    )(acc1, g1, dis, b1[None, :], W2)

    acc2 = _agg64(g2, idx4, zeros_o)

    out = pl.pallas_call(
        _t3_body,
        grid=grid,
        in_specs=[
            pl.BlockSpec((NC, BN, OUT_C), lambda i: (0, i, 0)),
            pl.BlockSpec((BN, OUT_C), lambda i: (i, 0)),
            pl.BlockSpec((BN, 1), lambda i: (i, 0)),
            pl.BlockSpec((1, OUT_C), lambda i: (0, 0)),
        ],
        out_specs=pl.BlockSpec((BN, OUT_C), lambda i: (i, 0)),
        out_shape=jax.ShapeDtypeStruct((NPAD, OUT_C), jnp.float32),
    )(acc2, g2, dis, b2[None, :])

    return out[:N_NODES]
